# baseline (speedup 1.0000x reference)
import functools

import jax
import jax.numpy as jnp
from jax import lax
from jax.experimental import pallas as pl
from jax.experimental.pallas import tpu as pltpu
from jax.experimental.pallas import tpu_sc as plsc

DEPTH = 3
H = 128

NC = 2
NS = 16
NW = NC * NS
CH = 128


def _init_body(fb_ref, w_ref, binput_ref, msg_ref):
    b = jnp.dot(fb_ref[...], w_ref[...], preferred_element_type=jnp.float32)
    binput_ref[...] = b.astype(jnp.bfloat16)
    msg_ref[...] = jnp.maximum(b, 0.0)


def _msg_update_body(nei_ref, bin_ref, w_ref, msg_ref):
    x = jnp.dot(nei_ref[...], w_ref[...], preferred_element_type=jnp.float32)
    msg_ref[...] = jnp.maximum(bin_ref[...].astype(jnp.float32) + x, 0.0)


def _atom_body(fa_ref, am_ref, wa_ref, wm_ref, b_ref, out_ref):
    h = jnp.dot(fa_ref[...], wa_ref[...], preferred_element_type=jnp.float32)
    h = h + jnp.dot(am_ref[...], wm_ref[...], preferred_element_type=jnp.float32)
    h = jnp.maximum(h + b_ref[...], 0.0)
    rows = h.shape[0]
    mols = out_ref.shape[0]
    apm = rows // mols
    r = lax.broadcasted_iota(jnp.int32, (mols, rows), 1)
    m = lax.broadcasted_iota(jnp.int32, (mols, rows), 0)
    sel = (r // apm == m).astype(jnp.float32)
    out_ref[...] = jnp.dot(sel, h, preferred_element_type=jnp.float32) * (1.0 / apm)


def _make_gather_sum(k, n_chunks, ch=128, nb=2, pack_bf16=False):
    per_w = nb * (-(-(-(-n_chunks // NW)) // nb))
    n_groups = per_w // nb
    mesh = plsc.VectorSubcoreMesh(core_axis_name="c", subcore_axis_name="s")
    out_cols = H // 2 if pack_bf16 else H
    out_dtype = jnp.int32 if pack_bf16 else jnp.float32

    def body(table_hbm, idx_hbm, out_hbm, *scratch):
        idx_v = scratch[0:nb]
        dst_v = scratch[nb:2 * nb]
        pk_v = scratch[2 * nb:3 * nb] if pack_bf16 else dst_v
        off = 3 * nb if pack_bf16 else 2 * nb
        sem_i = scratch[off:off + nb]
        sem_g = scratch[off + nb:off + 2 * nb]
        sem_s = scratch[off + 2 * nb:off + 3 * nb]
        wid = lax.axis_index("s") * NC + lax.axis_index("c")

        def fire_idx(c, b):
            pltpu.async_copy(idx_hbm.at[pl.ds(c * (k * ch), k * ch)],
                             idx_v[b], sem_i[b])

        def wait_idx(b):
            pltpu.make_async_copy(idx_hbm.at[pl.ds(0, k * ch)],
                                  idx_v[b], sem_i[b]).wait()

        def wait_store(b):
            pltpu.make_async_copy(pk_v[b], out_hbm.at[pl.ds(0, ch)],
                                  sem_s[b]).wait()

        for b in range(nb):
            @pl.when(wid + NW * b < n_chunks)
            def _(b=b):
                fire_idx(wid + NW * b, b)

        def group(p, carry):
            def chunk_id(b):
                return wid + NW * (nb * p + b)

            for b in range(nb):
                @pl.when(chunk_id(b) < n_chunks)
                def _(b=b):
                    @pl.when(p > 0)
                    def _():
                        wait_store(b)
                    wait_idx(b)

            for b in range(nb):
                @pl.when(chunk_id(b) < n_chunks)
                def _(b=b):
                    pltpu.async_copy(
                        table_hbm.at[idx_v[b].at[pl.ds(0, ch)]],
                        dst_v[b], sem_g[b])

            for b in range(nb):
                @pl.when(chunk_id(b) < n_chunks)
                def _(b=b):
                    pltpu.make_async_copy(
                        table_hbm.at[idx_v[b].at[pl.ds(0, ch)]],
                        dst_v[b], sem_g[b]).wait()
                    for j in range(1, k):
                        pltpu.async_copy(
                            table_hbm.at[idx_v[b].at[pl.ds(j * ch, ch)]],
                            dst_v[b], sem_g[b], add=True)

            for b in range(nb):
                c = chunk_id(b)

                @pl.when(c < n_chunks)
                def _(b=b, c=c):
                    for j in range(1, k):
                        pltpu.make_async_copy(
                            table_hbm.at[idx_v[b].at[pl.ds(j * ch, ch)]],
                            dst_v[b], sem_g[b]).wait()
                    if pack_bf16:
                        def pack_row(r, cr):
                            src = dst_v[b].at[r]
                            dst = pk_v[b].at[r]
                            for g in range(H // 32):
                                x = src[pl.ds(g * 32, 16)]
                                y = src[pl.ds(g * 32 + 16, 16)]
                                u = lax.bitcast_convert_type(x, jnp.int32)
                                v = lax.bitcast_convert_type(y, jnp.int32)
                                rx = u + 0x7FFF + ((u >> 16) & 1)
                                ry = v + 0x7FFF + ((v >> 16) & 1)
                                dst[pl.ds(g * 16, 16)] = (
                                    (ry & jnp.int32(-65536))
                                    | ((rx >> 16) & 0xFFFF))
                            return cr
                        lax.fori_loop(0, ch, pack_row, 0)
                    pltpu.async_copy(pk_v[b], out_hbm.at[pl.ds(c * ch, ch)],
                                     sem_s[b])

                    @pl.when(jnp.logical_and(p + 1 < n_groups,
                                             c + NW * nb < n_chunks))
                    def _():
                        fire_idx(c + NW * nb, b)

            return carry

        lax.fori_loop(0, n_groups, group, 0)

        for b in range(nb):
            @pl.when(wid + NW * b < n_chunks)
            def _(b=b):
                wait_store(b)

    return pl.kernel(
        body,
        out_type=jax.ShapeDtypeStruct((n_chunks * ch, out_cols), out_dtype),
        mesh=mesh,
        scratch_types=(
            [pltpu.VMEM((k * ch,), jnp.int32) for _ in range(nb)]
            + [pltpu.VMEM((ch, H), jnp.float32) for _ in range(nb)]
            + ([pltpu.VMEM((ch, H // 2), jnp.int32) for _ in range(nb)]
               if pack_bf16 else [])
            + [pltpu.SemaphoreType.DMA for _ in range(3 * nb)]
        ),
    )


def _pack_perm():
    perm = [0] * H
    for s in range(H // 32):
        for i in range(16):
            perm[32 * s + 2 * i] = 32 * s + i
            perm[32 * s + 2 * i + 1] = 32 * s + 16 + i
    return perm


def _chunked_idx(idx, n_chunks, ch=128):
    rows, k = idx.shape
    pad = n_chunks * ch - rows
    if pad:
        idx = jnp.pad(idx, ((0, pad), (0, 0)))
    return idx.reshape(n_chunks, ch, k).transpose(0, 2, 1).reshape(-1)


def kernel(fatoms, fbonds, agraph, bgraph, ascope, W_i, W_h, W_o_w, W_o_b):
    E, Fb = fbonds.shape
    N, Fa = fatoms.shape
    M = ascope.shape[0]
    kb = bgraph.shape[1]
    ka = agraph.shape[1]

    bgraph = bgraph.astype(jnp.int32)
    agraph = agraph.astype(jnp.int32)

    CHA = 64
    ncb = E // CH
    nca = -(-N // CHA)
    idx_b = _chunked_idx(bgraph, ncb, CH)
    idx_a = _chunked_idx(agraph, nca, CHA)

    BM1 = 3200
    binput, message = pl.pallas_call(
        _init_body,
        grid=(E // BM1,),
        in_specs=[pl.BlockSpec((BM1, Fb), lambda i: (i, 0)),
                  pl.BlockSpec((Fb, H), lambda i: (0, 0))],
        out_specs=[pl.BlockSpec((BM1, H), lambda i: (i, 0)),
                   pl.BlockSpec((BM1, H), lambda i: (i, 0))],
        out_shape=[jax.ShapeDtypeStruct((E, H), jnp.bfloat16),
                   jax.ShapeDtypeStruct((E, H), jnp.float32)],
    )(fbonds.astype(jnp.bfloat16), W_i.T.astype(jnp.bfloat16))

    gather_b = _make_gather_sum(kb, ncb, ch=CH, nb=3, pack_bf16=True)
    BM2 = 3200
    update = pl.pallas_call(
        _msg_update_body,
        grid=(E // BM2,),
        in_specs=[pl.BlockSpec((BM2, H), lambda i: (i, 0)),
                  pl.BlockSpec((BM2, H), lambda i: (i, 0)),
                  pl.BlockSpec((H, H), lambda i: (0, 0))],
        out_specs=pl.BlockSpec((BM2, H), lambda i: (i, 0)),
        out_shape=jax.ShapeDtypeStruct((E, H), jnp.float32),
    )
    W_hT = W_h.T[jnp.array(_pack_perm()), :].astype(jnp.bfloat16)
    for _ in range(DEPTH - 1):
        nei_packed = gather_b(message, idx_b)
        nei = lax.bitcast_convert_type(nei_packed, jnp.bfloat16).reshape(E, H)
        message = update(nei, binput, W_hT)

    gather_a = _make_gather_sum(ka, nca, ch=CHA, nb=2)
    a_msg = gather_a(message, idx_a)[:N]

    Wa_T = W_o_w[:, :Fa].T
    Wm_T = W_o_w[:, Fa:].T
    mol_vecs = pl.pallas_call(
        _atom_body,
        grid=(1,),
        in_specs=[pl.BlockSpec((N, Fa), lambda i: (0, 0)),
                  pl.BlockSpec((N, H), lambda i: (0, 0)),
                  pl.BlockSpec((Fa, H), lambda i: (0, 0)),
                  pl.BlockSpec((H, H), lambda i: (0, 0)),
                  pl.BlockSpec((1, H), lambda i: (0, 0))],
        out_specs=pl.BlockSpec((M, H), lambda i: (0, 0)),
        out_shape=jax.ShapeDtypeStruct((M, H), jnp.float32),
    )(fatoms, a_msg, Wa_T, Wm_T, W_o_b.reshape(1, H))
    return mol_vecs


# speedup vs baseline: 2.0447x; 2.0447x over previous
import functools

import jax
import jax.numpy as jnp
from jax import lax
from jax.experimental import pallas as pl
from jax.experimental.pallas import tpu as pltpu
from jax.experimental.pallas import tpu_sc as plsc

DEPTH = 3
H = 128

NC = 2
NS = 16
NW = NC * NS
CH = 128


def _init_body(fb_ref, w_ref, binput_ref, msg_ref):
    b = jnp.dot(fb_ref[...], w_ref[...], preferred_element_type=jnp.float32)
    binput_ref[...] = b.astype(jnp.bfloat16)
    msg_ref[...] = jnp.maximum(b, 0.0)


def _msg_update_body(nei_ref, bin_ref, w_ref, msg_ref):
    x = jnp.dot(nei_ref[...], w_ref[...], preferred_element_type=jnp.float32)
    msg_ref[...] = jnp.maximum(bin_ref[...].astype(jnp.float32) + x, 0.0)


def _atom_body(fa_ref, am_ref, wa_ref, wm_ref, b_ref, out_ref):
    h = jnp.dot(fa_ref[...], wa_ref[...], preferred_element_type=jnp.float32)
    h = h + jnp.dot(am_ref[...], wm_ref[...], preferred_element_type=jnp.float32)
    h = jnp.maximum(h + b_ref[...], 0.0)
    rows = h.shape[0]
    mols = out_ref.shape[0]
    apm = rows // mols
    r = lax.broadcasted_iota(jnp.int32, (mols, rows), 1)
    m = lax.broadcasted_iota(jnp.int32, (mols, rows), 0)
    sel = (r // apm == m).astype(jnp.float32)
    out_ref[...] = jnp.dot(sel, h, preferred_element_type=jnp.float32) * (1.0 / apm)


def _make_gather_sum(k, n_chunks, ch=128, nb=2, pack_bf16=False):
    per_w = nb * (-(-(-(-n_chunks // NW)) // nb))
    n_groups = per_w // nb
    mesh = plsc.VectorSubcoreMesh(core_axis_name="c", subcore_axis_name="s")
    out_cols = H // 2 if pack_bf16 else H
    out_dtype = jnp.int32 if pack_bf16 else jnp.float32

    def body(table_hbm, idx_hbm, out_hbm, *scratch):
        idx_v = scratch[0:nb]
        dst_v = scratch[nb:2 * nb]
        pk_v = scratch[2 * nb:3 * nb] if pack_bf16 else dst_v
        off = 3 * nb if pack_bf16 else 2 * nb
        sem_i = scratch[off:off + nb]
        sem_g = scratch[off + nb:off + 2 * nb]
        sem_s = scratch[off + 2 * nb:off + 3 * nb]
        wid = lax.axis_index("s") * NC + lax.axis_index("c")

        def fire_idx(c, b):
            pltpu.async_copy(idx_hbm.at[pl.ds(c * (k * ch), k * ch)],
                             idx_v[b], sem_i[b])

        def wait_idx(b):
            pltpu.make_async_copy(idx_hbm.at[pl.ds(0, k * ch)],
                                  idx_v[b], sem_i[b]).wait()

        def wait_store(b):
            pltpu.make_async_copy(pk_v[b], out_hbm.at[pl.ds(0, ch)],
                                  sem_s[b]).wait()

        for b in range(nb):
            @pl.when(wid + NW * b < n_chunks)
            def _(b=b):
                fire_idx(wid + NW * b, b)

        def group(p, carry):
            def chunk_id(b):
                return wid + NW * (nb * p + b)

            for b in range(nb):
                @pl.when(chunk_id(b) < n_chunks)
                def _(b=b):
                    @pl.when(p > 0)
                    def _():
                        wait_store(b)
                    wait_idx(b)

            for b in range(nb):
                @pl.when(chunk_id(b) < n_chunks)
                def _(b=b):
                    pltpu.async_copy(
                        table_hbm.at[idx_v[b].at[pl.ds(0, ch)]],
                        dst_v[b], sem_g[b])

            for b in range(nb):
                @pl.when(chunk_id(b) < n_chunks)
                def _(b=b):
                    pltpu.make_async_copy(
                        table_hbm.at[idx_v[b].at[pl.ds(0, ch)]],
                        dst_v[b], sem_g[b]).wait()
                    for j in range(1, k):
                        pltpu.async_copy(
                            table_hbm.at[idx_v[b].at[pl.ds(j * ch, ch)]],
                            dst_v[b], sem_g[b], add=True)

            for b in range(nb):
                c = chunk_id(b)

                @pl.when(c < n_chunks)
                def _(b=b, c=c):
                    for j in range(1, k):
                        pltpu.make_async_copy(
                            table_hbm.at[idx_v[b].at[pl.ds(j * ch, ch)]],
                            dst_v[b], sem_g[b]).wait()
                    if pack_bf16:
                        def pack_row(r, cr):
                            src = dst_v[b].at[r]
                            dst = pk_v[b].at[r]
                            for g in range(H // 32):
                                x = src[pl.ds(g * 32, 16)]
                                y = src[pl.ds(g * 32 + 16, 16)]
                                u = lax.bitcast_convert_type(x, jnp.int32)
                                v = lax.bitcast_convert_type(y, jnp.int32)
                                rx = u + 0x7FFF + ((u >> 16) & 1)
                                ry = v + 0x7FFF + ((v >> 16) & 1)
                                dst[pl.ds(g * 16, 16)] = (
                                    (ry & jnp.int32(-65536))
                                    | ((rx >> 16) & 0xFFFF))
                            return cr
                        lax.fori_loop(0, ch, pack_row, 0)
                    pltpu.async_copy(pk_v[b], out_hbm.at[pl.ds(c * ch, ch)],
                                     sem_s[b])

                    @pl.when(jnp.logical_and(p + 1 < n_groups,
                                             c + NW * nb < n_chunks))
                    def _():
                        fire_idx(c + NW * nb, b)

            return carry

        lax.fori_loop(0, n_groups, group, 0)

        for b in range(nb):
            @pl.when(wid + NW * b < n_chunks)
            def _(b=b):
                wait_store(b)

    return pl.kernel(
        body,
        out_type=jax.ShapeDtypeStruct((n_chunks * ch, out_cols), out_dtype),
        mesh=mesh,
        scratch_types=(
            [pltpu.VMEM((k * ch,), jnp.int32) for _ in range(nb)]
            + [pltpu.VMEM((ch, H), jnp.float32) for _ in range(nb)]
            + ([pltpu.VMEM((ch, H // 2), jnp.int32) for _ in range(nb)]
               if pack_bf16 else [])
            + [pltpu.SemaphoreType.DMA for _ in range(3 * nb)]
        ),
    )


def _pack_perm():
    perm = [0] * H
    for s in range(H // 32):
        for i in range(16):
            perm[32 * s + 2 * i] = 32 * s + i
            perm[32 * s + 2 * i + 1] = 32 * s + 16 + i
    return perm


def _chunked_idx(idx, n_chunks, ch=128):
    rows, k = idx.shape
    pad = n_chunks * ch - rows
    if pad:
        idx = jnp.pad(idx, ((0, pad), (0, 0)))
    return idx.reshape(n_chunks, ch, k).transpose(0, 2, 1).reshape(-1)


def kernel(fatoms, fbonds, agraph, bgraph, ascope, W_i, W_h, W_o_w, W_o_b):
    E, Fb = fbonds.shape
    N, Fa = fatoms.shape
    M = ascope.shape[0]
    kb = bgraph.shape[1]
    ka = agraph.shape[1]

    bgraph = bgraph.astype(jnp.int32)
    agraph = agraph.astype(jnp.int32)

    CHA = 64
    ncb = E // CH
    nca = -(-N // CHA)
    idx_b = _chunked_idx(bgraph, ncb, CH)
    idx_a = _chunked_idx(agraph, nca, CHA)

    BM1 = 3200
    binput, message = pl.pallas_call(
        _init_body,
        grid=(E // BM1,),
        in_specs=[pl.BlockSpec((BM1, Fb), lambda i: (i, 0)),
                  pl.BlockSpec((Fb, H), lambda i: (0, 0))],
        out_specs=[pl.BlockSpec((BM1, H), lambda i: (i, 0)),
                   pl.BlockSpec((BM1, H), lambda i: (i, 0))],
        out_shape=[jax.ShapeDtypeStruct((E, H), jnp.bfloat16),
                   jax.ShapeDtypeStruct((E, H), jnp.float32)],
    )(fbonds.astype(jnp.bfloat16), W_i.T.astype(jnp.bfloat16))

    gather_b = _make_gather_sum(kb, ncb, ch=CH, nb=6)
    BM2 = 3200
    update = pl.pallas_call(
        _msg_update_body,
        grid=(E // BM2,),
        in_specs=[pl.BlockSpec((BM2, H), lambda i: (i, 0)),
                  pl.BlockSpec((BM2, H), lambda i: (i, 0)),
                  pl.BlockSpec((H, H), lambda i: (0, 0))],
        out_specs=pl.BlockSpec((BM2, H), lambda i: (i, 0)),
        out_shape=jax.ShapeDtypeStruct((E, H), jnp.float32),
    )
    W_hT = W_h.T
    for _ in range(DEPTH - 1):
        nei = gather_b(message, idx_b)
        message = update(nei, binput, W_hT)

    gather_a = _make_gather_sum(ka, nca, ch=CHA, nb=2)
    a_msg = gather_a(message, idx_a)[:N]

    Wa_T = W_o_w[:, :Fa].T
    Wm_T = W_o_w[:, Fa:].T
    mol_vecs = pl.pallas_call(
        _atom_body,
        grid=(1,),
        in_specs=[pl.BlockSpec((N, Fa), lambda i: (0, 0)),
                  pl.BlockSpec((N, H), lambda i: (0, 0)),
                  pl.BlockSpec((Fa, H), lambda i: (0, 0)),
                  pl.BlockSpec((H, H), lambda i: (0, 0)),
                  pl.BlockSpec((1, H), lambda i: (0, 0))],
        out_specs=pl.BlockSpec((M, H), lambda i: (0, 0)),
        out_shape=jax.ShapeDtypeStruct((M, H), jnp.float32),
    )(fatoms, a_msg, Wa_T, Wm_T, W_o_b.reshape(1, H))
    return mol_vecs


# speedup vs baseline: 2.0539x; 1.0045x over previous
import functools

import jax
import jax.numpy as jnp
from jax import lax
from jax.experimental import pallas as pl
from jax.experimental.pallas import tpu as pltpu
from jax.experimental.pallas import tpu_sc as plsc

DEPTH = 3
H = 128

NC = 2
NS = 16
NW = NC * NS
CH = 128


def _init_body(fb_ref, w_ref, binput_ref, msg_ref):
    b = jnp.dot(fb_ref[...], w_ref[...], preferred_element_type=jnp.float32)
    binput_ref[...] = b.astype(jnp.bfloat16)
    msg_ref[...] = jnp.maximum(b, 0.0)


def _msg_update_body(nei_ref, bin_ref, w_ref, msg_ref):
    x = jnp.dot(nei_ref[...], w_ref[...], preferred_element_type=jnp.float32)
    msg_ref[...] = jnp.maximum(bin_ref[...].astype(jnp.float32) + x, 0.0)


def _atom_body(fa_ref, am_ref, wa_ref, wm_ref, b_ref, out_ref):
    h = jnp.dot(fa_ref[...], wa_ref[...], preferred_element_type=jnp.float32)
    h = h + jnp.dot(am_ref[...], wm_ref[...], preferred_element_type=jnp.float32)
    h = jnp.maximum(h + b_ref[...], 0.0)
    rows = h.shape[0]
    mols = out_ref.shape[0]
    apm = rows // mols
    r = lax.broadcasted_iota(jnp.int32, (mols, rows), 1)
    m = lax.broadcasted_iota(jnp.int32, (mols, rows), 0)
    sel = (r // apm == m).astype(jnp.float32)
    out_ref[...] = jnp.dot(sel, h, preferred_element_type=jnp.float32) * (1.0 / apm)


def _make_gather_sum(k, n_chunks, ch=128, nb=2, pack_bf16=False):
    per_w = nb * (-(-(-(-n_chunks // NW)) // nb))
    n_groups = per_w // nb
    mesh = plsc.VectorSubcoreMesh(core_axis_name="c", subcore_axis_name="s")
    out_cols = H // 2 if pack_bf16 else H
    out_dtype = jnp.int32 if pack_bf16 else jnp.float32

    def body(table_hbm, idx_hbm, out_hbm, *scratch):
        idx_v = scratch[0:nb]
        dst_v = scratch[nb:2 * nb]
        pk_v = scratch[2 * nb:3 * nb] if pack_bf16 else dst_v
        off = 3 * nb if pack_bf16 else 2 * nb
        sem_i = scratch[off:off + nb]
        sem_g = scratch[off + nb:off + 2 * nb]
        sem_s = scratch[off + 2 * nb:off + 3 * nb]
        wid = lax.axis_index("s") * NC + lax.axis_index("c")

        def fire_idx(c, b):
            pltpu.async_copy(idx_hbm.at[pl.ds(c * (k * ch), k * ch)],
                             idx_v[b], sem_i[b])

        def wait_idx(b):
            pltpu.make_async_copy(idx_hbm.at[pl.ds(0, k * ch)],
                                  idx_v[b], sem_i[b]).wait()

        def wait_store(b):
            pltpu.make_async_copy(pk_v[b], out_hbm.at[pl.ds(0, ch)],
                                  sem_s[b]).wait()

        for b in range(nb):
            @pl.when(wid + NW * b < n_chunks)
            def _(b=b):
                fire_idx(wid + NW * b, b)

        def group(p, carry):
            def chunk_id(b):
                return wid + NW * (nb * p + b)

            for b in range(nb):
                @pl.when(chunk_id(b) < n_chunks)
                def _(b=b):
                    @pl.when(p > 0)
                    def _():
                        wait_store(b)
                    wait_idx(b)

            for b in range(nb):
                @pl.when(chunk_id(b) < n_chunks)
                def _(b=b):
                    pltpu.async_copy(
                        table_hbm.at[idx_v[b].at[pl.ds(0, ch)]],
                        dst_v[b], sem_g[b])

            for b in range(nb):
                @pl.when(chunk_id(b) < n_chunks)
                def _(b=b):
                    pltpu.make_async_copy(
                        table_hbm.at[idx_v[b].at[pl.ds(0, ch)]],
                        dst_v[b], sem_g[b]).wait()
                    for j in range(1, k):
                        pltpu.async_copy(
                            table_hbm.at[idx_v[b].at[pl.ds(j * ch, ch)]],
                            dst_v[b], sem_g[b], add=True)

            for b in range(nb):
                c = chunk_id(b)

                @pl.when(c < n_chunks)
                def _(b=b, c=c):
                    for j in range(1, k):
                        pltpu.make_async_copy(
                            table_hbm.at[idx_v[b].at[pl.ds(j * ch, ch)]],
                            dst_v[b], sem_g[b]).wait()
                    if pack_bf16:
                        def pack_row(r, cr):
                            src = dst_v[b].at[r]
                            dst = pk_v[b].at[r]
                            for g in range(H // 32):
                                x = src[pl.ds(g * 32, 16)]
                                y = src[pl.ds(g * 32 + 16, 16)]
                                u = lax.bitcast_convert_type(x, jnp.int32)
                                v = lax.bitcast_convert_type(y, jnp.int32)
                                rx = u + 0x7FFF + ((u >> 16) & 1)
                                ry = v + 0x7FFF + ((v >> 16) & 1)
                                dst[pl.ds(g * 16, 16)] = (
                                    (ry & jnp.int32(-65536))
                                    | ((rx >> 16) & 0xFFFF))
                            return cr
                        lax.fori_loop(0, ch, pack_row, 0)
                    pltpu.async_copy(pk_v[b], out_hbm.at[pl.ds(c * ch, ch)],
                                     sem_s[b])

                    @pl.when(jnp.logical_and(p + 1 < n_groups,
                                             c + NW * nb < n_chunks))
                    def _():
                        fire_idx(c + NW * nb, b)

            return carry

        lax.fori_loop(0, n_groups, group, 0)

        for b in range(nb):
            @pl.when(wid + NW * b < n_chunks)
            def _(b=b):
                wait_store(b)

    return pl.kernel(
        body,
        out_type=jax.ShapeDtypeStruct((n_chunks * ch, out_cols), out_dtype),
        mesh=mesh,
        scratch_types=(
            [pltpu.VMEM((k * ch,), jnp.int32) for _ in range(nb)]
            + [pltpu.VMEM((ch, H), jnp.float32) for _ in range(nb)]
            + ([pltpu.VMEM((ch, H // 2), jnp.int32) for _ in range(nb)]
               if pack_bf16 else [])
            + [pltpu.SemaphoreType.DMA for _ in range(3 * nb)]
        ),
    )


def _pack_perm():
    perm = [0] * H
    for s in range(H // 32):
        for i in range(16):
            perm[32 * s + 2 * i] = 32 * s + i
            perm[32 * s + 2 * i + 1] = 32 * s + 16 + i
    return perm


def _chunked_idx(idx, n_chunks, ch=128):
    rows, k = idx.shape
    pad = n_chunks * ch - rows
    if pad:
        idx = jnp.pad(idx, ((0, pad), (0, 0)))
    return idx.reshape(n_chunks, ch, k).transpose(0, 2, 1).reshape(-1)


def kernel(fatoms, fbonds, agraph, bgraph, ascope, W_i, W_h, W_o_w, W_o_b):
    E, Fb = fbonds.shape
    N, Fa = fatoms.shape
    M = ascope.shape[0]
    kb = bgraph.shape[1]
    ka = agraph.shape[1]

    bgraph = bgraph.astype(jnp.int32)
    agraph = agraph.astype(jnp.int32)

    CHA = 64
    ncb = E // CH
    nca = -(-N // CHA)
    idx_b = _chunked_idx(bgraph, ncb, CH)
    idx_a = _chunked_idx(agraph, nca, CHA)

    BM1 = 3200
    binput, message = pl.pallas_call(
        _init_body,
        grid=(E // BM1,),
        in_specs=[pl.BlockSpec((BM1, Fb), lambda i: (i, 0)),
                  pl.BlockSpec((Fb, H), lambda i: (0, 0))],
        out_specs=[pl.BlockSpec((BM1, H), lambda i: (i, 0)),
                   pl.BlockSpec((BM1, H), lambda i: (i, 0))],
        out_shape=[jax.ShapeDtypeStruct((E, H), jnp.bfloat16),
                   jax.ShapeDtypeStruct((E, H), jnp.float32)],
    )(fbonds.astype(jnp.bfloat16), W_i.T.astype(jnp.bfloat16))

    gather_b = _make_gather_sum(kb, ncb, ch=CH, nb=6)
    BM2 = 3200
    update = pl.pallas_call(
        _msg_update_body,
        grid=(E // BM2,),
        in_specs=[pl.BlockSpec((BM2, H), lambda i: (i, 0)),
                  pl.BlockSpec((BM2, H), lambda i: (i, 0)),
                  pl.BlockSpec((H, H), lambda i: (0, 0))],
        out_specs=pl.BlockSpec((BM2, H), lambda i: (i, 0)),
        out_shape=jax.ShapeDtypeStruct((E, H), jnp.float32),
    )
    W_hT = W_h.T
    for _ in range(DEPTH - 1):
        nei = gather_b(message, idx_b)
        message = update(nei, binput, W_hT)

    gather_a = _make_gather_sum(ka, nca, ch=CHA, nb=3)
    a_msg = gather_a(message, idx_a)[:N]

    Wa_T = W_o_w[:, :Fa].T
    Wm_T = W_o_w[:, Fa:].T
    mol_vecs = pl.pallas_call(
        _atom_body,
        grid=(1,),
        in_specs=[pl.BlockSpec((N, Fa), lambda i: (0, 0)),
                  pl.BlockSpec((N, H), lambda i: (0, 0)),
                  pl.BlockSpec((Fa, H), lambda i: (0, 0)),
                  pl.BlockSpec((H, H), lambda i: (0, 0)),
                  pl.BlockSpec((1, H), lambda i: (0, 0))],
        out_specs=pl.BlockSpec((M, H), lambda i: (0, 0)),
        out_shape=jax.ShapeDtypeStruct((M, H), jnp.float32),
    )(fatoms, a_msg, Wa_T, Wm_T, W_o_b.reshape(1, H))
    return mol_vecs


# speedup vs baseline: 2.1279x; 1.0360x over previous
import functools

import jax
import jax.numpy as jnp
from jax import lax
from jax.experimental import pallas as pl
from jax.experimental.pallas import tpu as pltpu
from jax.experimental.pallas import tpu_sc as plsc

DEPTH = 3
H = 128

NC = 2
NS = 16
NW = NC * NS
CH = 128


def _init_body(fb_ref, w_ref, binput_ref, msg_ref):
    b = jnp.dot(fb_ref[...], w_ref[...], preferred_element_type=jnp.float32)
    binput_ref[...] = b.astype(jnp.bfloat16)
    msg_ref[...] = jnp.maximum(b, 0.0)


def _msg_update_body(nei_ref, bin_ref, w_ref, msg_ref):
    x = jnp.dot(nei_ref[...], w_ref[...], preferred_element_type=jnp.float32)
    msg_ref[...] = jnp.maximum(bin_ref[...].astype(jnp.float32) + x, 0.0)


def _atom_body(fa_ref, am_ref, wa_ref, wm_ref, b_ref, out_ref):
    h = jnp.dot(fa_ref[...], wa_ref[...], preferred_element_type=jnp.float32)
    h = h + jnp.dot(am_ref[...], wm_ref[...], preferred_element_type=jnp.float32)
    h = jnp.maximum(h + b_ref[...], 0.0)
    rows = h.shape[0]
    mols = out_ref.shape[0]
    apm = rows // mols
    r = lax.broadcasted_iota(jnp.int32, (mols, rows), 1)
    m = lax.broadcasted_iota(jnp.int32, (mols, rows), 0)
    sel = (r // apm == m).astype(jnp.float32)
    out_ref[...] = jnp.dot(sel, h, preferred_element_type=jnp.float32) * (1.0 / apm)


def _make_gather_sum(k, n_chunks, ch=128, nb=2, pack_bf16=False):
    per_w = nb * (-(-(-(-n_chunks // NW)) // nb))
    n_groups = per_w // nb
    mesh = plsc.VectorSubcoreMesh(core_axis_name="c", subcore_axis_name="s")
    out_cols = H // 2 if pack_bf16 else H
    out_dtype = jnp.int32 if pack_bf16 else jnp.float32

    def body(table_hbm, idx_hbm, out_hbm, *scratch):
        idx_v = scratch[0:nb]
        dst_v = scratch[nb:2 * nb]
        pk_v = scratch[2 * nb:3 * nb] if pack_bf16 else dst_v
        off = 3 * nb if pack_bf16 else 2 * nb
        sem_i = scratch[off:off + nb]
        sem_g = scratch[off + nb:off + 2 * nb]
        sem_s = scratch[off + 2 * nb:off + 3 * nb]
        wid = lax.axis_index("s") * NC + lax.axis_index("c")

        def fire_idx(c, b):
            pltpu.async_copy(idx_hbm.at[pl.ds(c * (k * ch), k * ch)],
                             idx_v[b], sem_i[b])

        def wait_idx(b):
            pltpu.make_async_copy(idx_hbm.at[pl.ds(0, k * ch)],
                                  idx_v[b], sem_i[b]).wait()

        def wait_store(b):
            pltpu.make_async_copy(pk_v[b], out_hbm.at[pl.ds(0, ch)],
                                  sem_s[b]).wait()

        for b in range(nb):
            @pl.when(wid + NW * b < n_chunks)
            def _(b=b):
                fire_idx(wid + NW * b, b)

        def group(p, carry):
            def chunk_id(b):
                return wid + NW * (nb * p + b)

            for b in range(nb):
                @pl.when(chunk_id(b) < n_chunks)
                def _(b=b):
                    @pl.when(p > 0)
                    def _():
                        wait_store(b)
                    wait_idx(b)

            for b in range(nb):
                @pl.when(chunk_id(b) < n_chunks)
                def _(b=b):
                    pltpu.async_copy(
                        table_hbm.at[idx_v[b].at[pl.ds(0, ch)]],
                        dst_v[b], sem_g[b])

            for b in range(nb):
                @pl.when(chunk_id(b) < n_chunks)
                def _(b=b):
                    pltpu.make_async_copy(
                        table_hbm.at[idx_v[b].at[pl.ds(0, ch)]],
                        dst_v[b], sem_g[b]).wait()
                    for j in range(1, k):
                        pltpu.async_copy(
                            table_hbm.at[idx_v[b].at[pl.ds(j * ch, ch)]],
                            dst_v[b], sem_g[b], add=True)

            for b in range(nb):
                c = chunk_id(b)

                @pl.when(c < n_chunks)
                def _(b=b, c=c):
                    for j in range(1, k):
                        pltpu.make_async_copy(
                            table_hbm.at[idx_v[b].at[pl.ds(j * ch, ch)]],
                            dst_v[b], sem_g[b]).wait()
                    if pack_bf16:
                        def pack_row(r, cr):
                            src = dst_v[b].at[r]
                            dst = pk_v[b].at[r]
                            for g in range(H // 32):
                                x = src[pl.ds(g * 32, 16)]
                                y = src[pl.ds(g * 32 + 16, 16)]
                                u = lax.bitcast_convert_type(x, jnp.int32)
                                v = lax.bitcast_convert_type(y, jnp.int32)
                                rx = u + 0x7FFF + ((u >> 16) & 1)
                                ry = v + 0x7FFF + ((v >> 16) & 1)
                                dst[pl.ds(g * 16, 16)] = (
                                    (ry & jnp.int32(-65536))
                                    | ((rx >> 16) & 0xFFFF))
                            return cr
                        lax.fori_loop(0, ch, pack_row, 0)
                    pltpu.async_copy(pk_v[b], out_hbm.at[pl.ds(c * ch, ch)],
                                     sem_s[b])

                    @pl.when(jnp.logical_and(p + 1 < n_groups,
                                             c + NW * nb < n_chunks))
                    def _():
                        fire_idx(c + NW * nb, b)

            return carry

        lax.fori_loop(0, n_groups, group, 0)

        for b in range(nb):
            @pl.when(wid + NW * b < n_chunks)
            def _(b=b):
                wait_store(b)

    return pl.kernel(
        body,
        out_type=jax.ShapeDtypeStruct((n_chunks * ch, out_cols), out_dtype),
        mesh=mesh,
        scratch_types=(
            [pltpu.VMEM((k * ch,), jnp.int32) for _ in range(nb)]
            + [pltpu.VMEM((ch, H), jnp.float32) for _ in range(nb)]
            + ([pltpu.VMEM((ch, H // 2), jnp.int32) for _ in range(nb)]
               if pack_bf16 else [])
            + [pltpu.SemaphoreType.DMA for _ in range(3 * nb)]
        ),
    )


def _pack_perm():
    perm = [0] * H
    for s in range(H // 32):
        for i in range(16):
            perm[32 * s + 2 * i] = 32 * s + i
            perm[32 * s + 2 * i + 1] = 32 * s + 16 + i
    return perm


def _chunked_idx(idx, n_chunks, ch=128):
    rows, k = idx.shape
    pad = n_chunks * ch - rows
    if pad:
        idx = jnp.pad(idx, ((0, pad), (0, 0)))
    return idx.reshape(n_chunks, ch, k).transpose(0, 2, 1).reshape(-1)


def kernel(fatoms, fbonds, agraph, bgraph, ascope, W_i, W_h, W_o_w, W_o_b):
    E, Fb = fbonds.shape
    N, Fa = fatoms.shape
    M = ascope.shape[0]
    kb = bgraph.shape[1]
    ka = agraph.shape[1]

    bgraph = bgraph.astype(jnp.int32)
    agraph = agraph.astype(jnp.int32)

    CHA = 64
    ncb = E // CH
    nca = -(-N // CHA)
    idx_b = _chunked_idx(bgraph, ncb, CH)
    idx_a = _chunked_idx(agraph, nca, CHA)

    BM1 = 6400
    binput, message = pl.pallas_call(
        _init_body,
        grid=(E // BM1,),
        in_specs=[pl.BlockSpec((BM1, Fb), lambda i: (i, 0)),
                  pl.BlockSpec((Fb, H), lambda i: (0, 0))],
        out_specs=[pl.BlockSpec((BM1, H), lambda i: (i, 0)),
                   pl.BlockSpec((BM1, H), lambda i: (i, 0))],
        out_shape=[jax.ShapeDtypeStruct((E, H), jnp.bfloat16),
                   jax.ShapeDtypeStruct((E, H), jnp.float32)],
    )(fbonds.astype(jnp.bfloat16), W_i.T.astype(jnp.bfloat16))

    gather_b = _make_gather_sum(kb, ncb, ch=CH, nb=6)
    BM2 = 6400
    update = pl.pallas_call(
        _msg_update_body,
        grid=(E // BM2,),
        in_specs=[pl.BlockSpec((BM2, H), lambda i: (i, 0)),
                  pl.BlockSpec((BM2, H), lambda i: (i, 0)),
                  pl.BlockSpec((H, H), lambda i: (0, 0))],
        out_specs=pl.BlockSpec((BM2, H), lambda i: (i, 0)),
        out_shape=jax.ShapeDtypeStruct((E, H), jnp.float32),
    )
    W_hT = W_h.T
    for _ in range(DEPTH - 1):
        nei = gather_b(message, idx_b)
        message = update(nei, binput, W_hT)

    gather_a = _make_gather_sum(ka, nca, ch=CHA, nb=3)
    a_msg = gather_a(message, idx_a)[:N]

    Wa_T = W_o_w[:, :Fa].T
    Wm_T = W_o_w[:, Fa:].T
    mol_vecs = pl.pallas_call(
        _atom_body,
        grid=(1,),
        in_specs=[pl.BlockSpec((N, Fa), lambda i: (0, 0)),
                  pl.BlockSpec((N, H), lambda i: (0, 0)),
                  pl.BlockSpec((Fa, H), lambda i: (0, 0)),
                  pl.BlockSpec((H, H), lambda i: (0, 0)),
                  pl.BlockSpec((1, H), lambda i: (0, 0))],
        out_specs=pl.BlockSpec((M, H), lambda i: (0, 0)),
        out_shape=jax.ShapeDtypeStruct((M, H), jnp.float32),
    )(fatoms, a_msg, Wa_T, Wm_T, W_o_b.reshape(1, H))
    return mol_vecs


# speedup vs baseline: 2.1447x; 1.0079x over previous
import functools

import jax
import jax.numpy as jnp
from jax import lax
from jax.experimental import pallas as pl
from jax.experimental.pallas import tpu as pltpu
from jax.experimental.pallas import tpu_sc as plsc

DEPTH = 3
H = 128

NC = 2
NS = 16
NW = NC * NS
CH = 128


def _init_body(fb_ref, w_ref, binput_ref, msg_ref):
    b = jnp.dot(fb_ref[...], w_ref[...], preferred_element_type=jnp.float32)
    binput_ref[...] = b.astype(jnp.bfloat16)
    msg_ref[...] = jnp.maximum(b, 0.0)


def _msg_update_body(nei_ref, bin_ref, w_ref, msg_ref):
    x = jnp.dot(nei_ref[...], w_ref[...], preferred_element_type=jnp.float32)
    msg_ref[...] = jnp.maximum(bin_ref[...].astype(jnp.float32) + x, 0.0)


def _atom_body(fa_ref, am_ref, wa_ref, wm_ref, b_ref, out_ref):
    h = jnp.dot(fa_ref[...], wa_ref[...], preferred_element_type=jnp.float32)
    h = h + jnp.dot(am_ref[...], wm_ref[...], preferred_element_type=jnp.float32)
    h = jnp.maximum(h + b_ref[...], 0.0)
    rows = h.shape[0]
    mols = out_ref.shape[0]
    apm = rows // mols
    r = lax.broadcasted_iota(jnp.int32, (mols, rows), 1)
    m = lax.broadcasted_iota(jnp.int32, (mols, rows), 0)
    sel = (r // apm == m).astype(jnp.float32)
    out_ref[...] = jnp.dot(sel, h, preferred_element_type=jnp.float32) * (1.0 / apm)


def _make_gather_sum(k, n_chunks, ch=128, nb=2, pack_bf16=False):
    per_w = nb * (-(-(-(-n_chunks // NW)) // nb))
    n_groups = per_w // nb
    mesh = plsc.VectorSubcoreMesh(core_axis_name="c", subcore_axis_name="s")
    out_cols = H // 2 if pack_bf16 else H
    out_dtype = jnp.int32 if pack_bf16 else jnp.float32

    def body(table_hbm, idx_hbm, out_hbm, *scratch):
        idx_v = scratch[0:nb]
        dst_v = scratch[nb:2 * nb]
        pk_v = scratch[2 * nb:3 * nb] if pack_bf16 else dst_v
        off = 3 * nb if pack_bf16 else 2 * nb
        sem_i = scratch[off:off + nb]
        sem_g = scratch[off + nb:off + 2 * nb]
        sem_s = scratch[off + 2 * nb:off + 3 * nb]
        wid = lax.axis_index("s") * NC + lax.axis_index("c")

        def fire_idx(c, b):
            pltpu.async_copy(idx_hbm.at[pl.ds(c * (k * ch), k * ch)],
                             idx_v[b], sem_i[b])

        def wait_idx(b):
            pltpu.make_async_copy(idx_hbm.at[pl.ds(0, k * ch)],
                                  idx_v[b], sem_i[b]).wait()

        def wait_store(b):
            pltpu.make_async_copy(pk_v[b], out_hbm.at[pl.ds(0, ch)],
                                  sem_s[b]).wait()

        for b in range(nb):
            @pl.when(wid + NW * b < n_chunks)
            def _(b=b):
                fire_idx(wid + NW * b, b)

        def group(p, carry):
            def chunk_id(b):
                return wid + NW * (nb * p + b)

            for b in range(nb):
                @pl.when(chunk_id(b) < n_chunks)
                def _(b=b):
                    @pl.when(p > 0)
                    def _():
                        wait_store(b)
                    wait_idx(b)

            for b in range(nb):
                @pl.when(chunk_id(b) < n_chunks)
                def _(b=b):
                    pltpu.async_copy(
                        table_hbm.at[idx_v[b].at[pl.ds(0, ch)]],
                        dst_v[b], sem_g[b])

            for b in range(nb):
                @pl.when(chunk_id(b) < n_chunks)
                def _(b=b):
                    pltpu.make_async_copy(
                        table_hbm.at[idx_v[b].at[pl.ds(0, ch)]],
                        dst_v[b], sem_g[b]).wait()
                    for j in range(1, k):
                        pltpu.async_copy(
                            table_hbm.at[idx_v[b].at[pl.ds(j * ch, ch)]],
                            dst_v[b], sem_g[b], add=True)

            for b in range(nb):
                c = chunk_id(b)

                @pl.when(c < n_chunks)
                def _(b=b, c=c):
                    for j in range(1, k):
                        pltpu.make_async_copy(
                            table_hbm.at[idx_v[b].at[pl.ds(j * ch, ch)]],
                            dst_v[b], sem_g[b]).wait()
                    if pack_bf16:
                        def pack_row(r, cr):
                            src = dst_v[b].at[r]
                            dst = pk_v[b].at[r]
                            for g in range(H // 32):
                                x = src[pl.ds(g * 32, 16)]
                                y = src[pl.ds(g * 32 + 16, 16)]
                                u = lax.bitcast_convert_type(x, jnp.int32)
                                v = lax.bitcast_convert_type(y, jnp.int32)
                                rx = u + 0x7FFF + ((u >> 16) & 1)
                                ry = v + 0x7FFF + ((v >> 16) & 1)
                                dst[pl.ds(g * 16, 16)] = (
                                    (ry & jnp.int32(-65536))
                                    | ((rx >> 16) & 0xFFFF))
                            return cr
                        lax.fori_loop(0, ch, pack_row, 0)
                    pltpu.async_copy(pk_v[b], out_hbm.at[pl.ds(c * ch, ch)],
                                     sem_s[b])

                    @pl.when(jnp.logical_and(p + 1 < n_groups,
                                             c + NW * nb < n_chunks))
                    def _():
                        fire_idx(c + NW * nb, b)

            return carry

        lax.fori_loop(0, n_groups, group, 0)

        for b in range(nb):
            @pl.when(wid + NW * b < n_chunks)
            def _(b=b):
                wait_store(b)

    return pl.kernel(
        body,
        out_type=jax.ShapeDtypeStruct((n_chunks * ch, out_cols), out_dtype),
        mesh=mesh,
        scratch_types=(
            [pltpu.VMEM((k * ch,), jnp.int32) for _ in range(nb)]
            + [pltpu.VMEM((ch, H), jnp.float32) for _ in range(nb)]
            + ([pltpu.VMEM((ch, H // 2), jnp.int32) for _ in range(nb)]
               if pack_bf16 else [])
            + [pltpu.SemaphoreType.DMA for _ in range(3 * nb)]
        ),
    )


def _pack_perm():
    perm = [0] * H
    for s in range(H // 32):
        for i in range(16):
            perm[32 * s + 2 * i] = 32 * s + i
            perm[32 * s + 2 * i + 1] = 32 * s + 16 + i
    return perm


def _chunked_idx(idx, n_chunks, ch=128):
    rows, k = idx.shape
    pad = n_chunks * ch - rows
    if pad:
        idx = jnp.pad(idx, ((0, pad), (0, 0)))
    return idx.reshape(n_chunks, ch, k).transpose(0, 2, 1).reshape(-1)


def kernel(fatoms, fbonds, agraph, bgraph, ascope, W_i, W_h, W_o_w, W_o_b):
    E, Fb = fbonds.shape
    N, Fa = fatoms.shape
    M = ascope.shape[0]
    kb = bgraph.shape[1]
    ka = agraph.shape[1]

    bgraph = bgraph.astype(jnp.int32)
    agraph = agraph.astype(jnp.int32)

    CHA = 64
    ncb = E // CH
    nca = -(-N // CHA)
    idx_b = _chunked_idx(bgraph, ncb, CH)
    idx_a = _chunked_idx(agraph, nca, CHA)

    BM1 = 16000
    binput, message = pl.pallas_call(
        _init_body,
        grid=(E // BM1,),
        in_specs=[pl.BlockSpec((BM1, Fb), lambda i: (i, 0)),
                  pl.BlockSpec((Fb, H), lambda i: (0, 0))],
        out_specs=[pl.BlockSpec((BM1, H), lambda i: (i, 0)),
                   pl.BlockSpec((BM1, H), lambda i: (i, 0))],
        out_shape=[jax.ShapeDtypeStruct((E, H), jnp.bfloat16),
                   jax.ShapeDtypeStruct((E, H), jnp.float32)],
    )(fbonds.astype(jnp.bfloat16), W_i.T.astype(jnp.bfloat16))

    gather_b = _make_gather_sum(kb, ncb, ch=CH, nb=6)
    BM2 = 16000
    update = pl.pallas_call(
        _msg_update_body,
        grid=(E // BM2,),
        in_specs=[pl.BlockSpec((BM2, H), lambda i: (i, 0)),
                  pl.BlockSpec((BM2, H), lambda i: (i, 0)),
                  pl.BlockSpec((H, H), lambda i: (0, 0))],
        out_specs=pl.BlockSpec((BM2, H), lambda i: (i, 0)),
        out_shape=jax.ShapeDtypeStruct((E, H), jnp.float32),
    )
    W_hT = W_h.T
    for _ in range(DEPTH - 1):
        nei = gather_b(message, idx_b)
        message = update(nei, binput, W_hT)

    gather_a = _make_gather_sum(ka, nca, ch=CHA, nb=3)
    a_msg = gather_a(message, idx_a)[:N]

    Wa_T = W_o_w[:, :Fa].T
    Wm_T = W_o_w[:, Fa:].T
    mol_vecs = pl.pallas_call(
        _atom_body,
        grid=(1,),
        in_specs=[pl.BlockSpec((N, Fa), lambda i: (0, 0)),
                  pl.BlockSpec((N, H), lambda i: (0, 0)),
                  pl.BlockSpec((Fa, H), lambda i: (0, 0)),
                  pl.BlockSpec((H, H), lambda i: (0, 0)),
                  pl.BlockSpec((1, H), lambda i: (0, 0))],
        out_specs=pl.BlockSpec((M, H), lambda i: (0, 0)),
        out_shape=jax.ShapeDtypeStruct((M, H), jnp.float32),
    )(fatoms, a_msg, Wa_T, Wm_T, W_o_b.reshape(1, H))
    return mol_vecs


# speedup vs baseline: 2.1468x; 1.0010x over previous
import jax
import jax.numpy as jnp
from jax import lax
from jax.experimental import pallas as pl
from jax.experimental.pallas import tpu as pltpu
from jax.experimental.pallas import tpu_sc as plsc

DEPTH = 3
H = 128

NC = 2
NS = 16
NW = NC * NS
CH = 128


def _init_body(fb_ref, w_ref, binput_ref, msg_ref):
    b = jnp.dot(fb_ref[...], w_ref[...], preferred_element_type=jnp.float32)
    binput_ref[...] = b.astype(jnp.bfloat16)
    msg_ref[...] = jnp.maximum(b, 0.0)


def _msg_update_body(nei_ref, bin_ref, w_ref, msg_ref):
    x = jnp.dot(nei_ref[...], w_ref[...], preferred_element_type=jnp.float32)
    msg_ref[...] = jnp.maximum(bin_ref[...].astype(jnp.float32) + x, 0.0)


def _atom_body(fa_ref, am_ref, wa_ref, wm_ref, b_ref, out_ref):
    h = jnp.dot(fa_ref[...], wa_ref[...], preferred_element_type=jnp.float32)
    h = h + jnp.dot(am_ref[...], wm_ref[...], preferred_element_type=jnp.float32)
    h = jnp.maximum(h + b_ref[...], 0.0)
    rows = h.shape[0]
    mols = out_ref.shape[0]
    apm = rows // mols
    r = lax.broadcasted_iota(jnp.int32, (mols, rows), 1)
    m = lax.broadcasted_iota(jnp.int32, (mols, rows), 0)
    sel = (r // apm == m).astype(jnp.float32)
    out_ref[...] = jnp.dot(sel, h, preferred_element_type=jnp.float32) * (1.0 / apm)


def _make_gather_sum(k, n_chunks, ch=128, nb=2, pack_bf16=False):
    per_w = nb * (-(-(-(-n_chunks // NW)) // nb))
    n_groups = per_w // nb
    mesh = plsc.VectorSubcoreMesh(core_axis_name="c", subcore_axis_name="s")
    out_cols = H // 2 if pack_bf16 else H
    out_dtype = jnp.int32 if pack_bf16 else jnp.float32

    def body(table_hbm, idx_hbm, out_hbm, *scratch):
        idx_v = scratch[0:nb]
        dst_v = scratch[nb:2 * nb]
        pk_v = scratch[2 * nb:3 * nb] if pack_bf16 else dst_v
        off = 3 * nb if pack_bf16 else 2 * nb
        sem_i = scratch[off:off + nb]
        sem_g = scratch[off + nb:off + 2 * nb]
        sem_s = scratch[off + 2 * nb:off + 3 * nb]
        wid = lax.axis_index("s") * NC + lax.axis_index("c")

        def fire_idx(c, b):
            pltpu.async_copy(idx_hbm.at[pl.ds(c * (k * ch), k * ch)],
                             idx_v[b], sem_i[b])

        def wait_idx(b):
            pltpu.make_async_copy(idx_hbm.at[pl.ds(0, k * ch)],
                                  idx_v[b], sem_i[b]).wait()

        def wait_store(b):
            pltpu.make_async_copy(pk_v[b], out_hbm.at[pl.ds(0, ch)],
                                  sem_s[b]).wait()

        for b in range(nb):
            @pl.when(wid + NW * b < n_chunks)
            def _(b=b):
                fire_idx(wid + NW * b, b)

        def group(p, carry):
            def chunk_id(b):
                return wid + NW * (nb * p + b)

            for b in range(nb):
                @pl.when(chunk_id(b) < n_chunks)
                def _(b=b):
                    @pl.when(p > 0)
                    def _():
                        wait_store(b)
                    wait_idx(b)

            for b in range(nb):
                @pl.when(chunk_id(b) < n_chunks)
                def _(b=b):
                    pltpu.async_copy(
                        table_hbm.at[idx_v[b].at[pl.ds(0, ch)]],
                        dst_v[b], sem_g[b])

            for b in range(nb):
                @pl.when(chunk_id(b) < n_chunks)
                def _(b=b):
                    pltpu.make_async_copy(
                        table_hbm.at[idx_v[b].at[pl.ds(0, ch)]],
                        dst_v[b], sem_g[b]).wait()
                    for j in range(1, k):
                        pltpu.async_copy(
                            table_hbm.at[idx_v[b].at[pl.ds(j * ch, ch)]],
                            dst_v[b], sem_g[b], add=True)

            for b in range(nb):
                c = chunk_id(b)

                @pl.when(c < n_chunks)
                def _(b=b, c=c):
                    for j in range(1, k):
                        pltpu.make_async_copy(
                            table_hbm.at[idx_v[b].at[pl.ds(j * ch, ch)]],
                            dst_v[b], sem_g[b]).wait()
                    if pack_bf16:
                        def pack_row(r, cr):
                            src = dst_v[b].at[r]
                            dst = pk_v[b].at[r]
                            for g in range(H // 32):
                                x = src[pl.ds(g * 32, 16)]
                                y = src[pl.ds(g * 32 + 16, 16)]
                                u = lax.bitcast_convert_type(x, jnp.int32)
                                v = lax.bitcast_convert_type(y, jnp.int32)
                                rx = u + 0x7FFF + ((u >> 16) & 1)
                                ry = v + 0x7FFF + ((v >> 16) & 1)
                                dst[pl.ds(g * 16, 16)] = (
                                    (ry & jnp.int32(-65536))
                                    | ((rx >> 16) & 0xFFFF))
                            return cr
                        lax.fori_loop(0, ch, pack_row, 0)
                    pltpu.async_copy(pk_v[b], out_hbm.at[pl.ds(c * ch, ch)],
                                     sem_s[b])

                    @pl.when(jnp.logical_and(p + 1 < n_groups,
                                             c + NW * nb < n_chunks))
                    def _():
                        fire_idx(c + NW * nb, b)

            return carry

        lax.fori_loop(0, n_groups, group, 0)

        for b in range(nb):
            @pl.when(wid + NW * b < n_chunks)
            def _(b=b):
                wait_store(b)

    return pl.kernel(
        body,
        out_type=jax.ShapeDtypeStruct((n_chunks * ch, out_cols), out_dtype),
        mesh=mesh,
        scratch_types=(
            [pltpu.VMEM((k * ch,), jnp.int32) for _ in range(nb)]
            + [pltpu.VMEM((ch, H), jnp.float32) for _ in range(nb)]
            + ([pltpu.VMEM((ch, H // 2), jnp.int32) for _ in range(nb)]
               if pack_bf16 else [])
            + [pltpu.SemaphoreType.DMA for _ in range(3 * nb)]
        ),
    )


def _pack_perm():
    perm = [0] * H
    for s in range(H // 32):
        for i in range(16):
            perm[32 * s + 2 * i] = 32 * s + i
            perm[32 * s + 2 * i + 1] = 32 * s + 16 + i
    return perm


def _chunked_idx(idx, n_chunks, ch=128):
    rows, k = idx.shape
    pad = n_chunks * ch - rows
    if pad:
        idx = jnp.pad(idx, ((0, pad), (0, 0)))
    return idx.reshape(n_chunks, ch, k).transpose(0, 2, 1).reshape(-1)


def kernel(fatoms, fbonds, agraph, bgraph, ascope, W_i, W_h, W_o_w, W_o_b):
    E, Fb = fbonds.shape
    N, Fa = fatoms.shape
    M = ascope.shape[0]
    kb = bgraph.shape[1]
    ka = agraph.shape[1]

    bgraph = bgraph.astype(jnp.int32)
    agraph = agraph.astype(jnp.int32)

    CHA = 64
    ncb = E // CH
    nca = -(-N // CHA)
    idx_b = _chunked_idx(bgraph, ncb, CH)
    idx_a = _chunked_idx(agraph, nca, CHA)

    BM1 = 16000
    binput, message = pl.pallas_call(
        _init_body,
        grid=(E // BM1,),
        in_specs=[pl.BlockSpec((BM1, Fb), lambda i: (i, 0)),
                  pl.BlockSpec((Fb, H), lambda i: (0, 0))],
        out_specs=[pl.BlockSpec((BM1, H), lambda i: (i, 0)),
                   pl.BlockSpec((BM1, H), lambda i: (i, 0))],
        out_shape=[jax.ShapeDtypeStruct((E, H), jnp.bfloat16),
                   jax.ShapeDtypeStruct((E, H), jnp.float32)],
    )(fbonds.astype(jnp.bfloat16), W_i.T.astype(jnp.bfloat16))

    gather_b = _make_gather_sum(kb, ncb, ch=CH, nb=6)
    BM2 = 16000
    update = pl.pallas_call(
        _msg_update_body,
        grid=(E // BM2,),
        in_specs=[pl.BlockSpec((BM2, H), lambda i: (i, 0)),
                  pl.BlockSpec((BM2, H), lambda i: (i, 0)),
                  pl.BlockSpec((H, H), lambda i: (0, 0))],
        out_specs=pl.BlockSpec((BM2, H), lambda i: (i, 0)),
        out_shape=jax.ShapeDtypeStruct((E, H), jnp.float32),
    )
    W_hT = W_h.T
    for _ in range(DEPTH - 1):
        nei = gather_b(message, idx_b)
        message = update(nei, binput, W_hT)

    gather_a = _make_gather_sum(ka, nca, ch=CHA, nb=3)
    a_msg = gather_a(message, idx_a)[:N]

    Wa_T = W_o_w[:, :Fa].T
    Wm_T = W_o_w[:, Fa:].T
    mol_vecs = pl.pallas_call(
        _atom_body,
        grid=(1,),
        in_specs=[pl.BlockSpec((N, Fa), lambda i: (0, 0)),
                  pl.BlockSpec((N, H), lambda i: (0, 0)),
                  pl.BlockSpec((Fa, H), lambda i: (0, 0)),
                  pl.BlockSpec((H, H), lambda i: (0, 0)),
                  pl.BlockSpec((1, H), lambda i: (0, 0))],
        out_specs=pl.BlockSpec((M, H), lambda i: (0, 0)),
        out_shape=jax.ShapeDtypeStruct((M, H), jnp.float32),
    )(fatoms, a_msg, Wa_T, Wm_T, W_o_b.reshape(1, H))
    return mol_vecs


# speedup vs baseline: 2.1470x; 1.0001x over previous
import jax
import jax.numpy as jnp
from jax import lax
from jax.experimental import pallas as pl
from jax.experimental.pallas import tpu as pltpu
from jax.experimental.pallas import tpu_sc as plsc

DEPTH = 3
H = 128

NC = 2
NS = 16
NW = NC * NS
CH = 128


def _init_body(fb_ref, w_ref, binput_ref, msg_ref):
    b = jnp.dot(fb_ref[...], w_ref[...], preferred_element_type=jnp.float32)
    binput_ref[...] = b.astype(jnp.bfloat16)
    msg_ref[...] = jnp.maximum(b, 0.0)


def _msg_update_body(nei_ref, bin_ref, w_ref, msg_ref):
    x = jnp.dot(nei_ref[...], w_ref[...], preferred_element_type=jnp.float32)
    msg_ref[...] = jnp.maximum(bin_ref[...].astype(jnp.float32) + x, 0.0)


def _atom_body(fa_ref, am_ref, wa_ref, wm_ref, b_ref, out_ref):
    h = jnp.dot(fa_ref[...], wa_ref[...], preferred_element_type=jnp.float32)
    h = h + jnp.dot(am_ref[...], wm_ref[...], preferred_element_type=jnp.float32)
    h = jnp.maximum(h + b_ref[...], 0.0)
    rows = h.shape[0]
    mols = out_ref.shape[0]
    apm = rows // mols
    r = lax.broadcasted_iota(jnp.int32, (mols, rows), 1)
    m = lax.broadcasted_iota(jnp.int32, (mols, rows), 0)
    sel = (r // apm == m).astype(jnp.float32)
    out_ref[...] = jnp.dot(sel, h, preferred_element_type=jnp.float32) * (1.0 / apm)


def _make_gather_sum(k, n_chunks, ch=128, nb=2):
    per_w = nb * (-(-(-(-n_chunks // NW)) // nb))
    n_groups = per_w // nb
    mesh = plsc.VectorSubcoreMesh(core_axis_name="c", subcore_axis_name="s")

    def body(table_hbm, idx_hbm, out_hbm, *scratch):
        idx_v = scratch[0:nb]
        dst_v = scratch[nb:2 * nb]
        sem_i = scratch[2 * nb:3 * nb]
        sem_g = scratch[3 * nb:4 * nb]
        sem_s = scratch[4 * nb:5 * nb]
        wid = lax.axis_index("s") * NC + lax.axis_index("c")

        def fire_idx(c, b):
            pltpu.async_copy(idx_hbm.at[pl.ds(c * (k * ch), k * ch)],
                             idx_v[b], sem_i[b])

        def wait_idx(b):
            pltpu.make_async_copy(idx_hbm.at[pl.ds(0, k * ch)],
                                  idx_v[b], sem_i[b]).wait()

        def wait_store(b):
            pltpu.make_async_copy(dst_v[b], out_hbm.at[pl.ds(0, ch)],
                                  sem_s[b]).wait()

        for b in range(nb):
            @pl.when(wid + NW * b < n_chunks)
            def _(b=b):
                fire_idx(wid + NW * b, b)

        def group(p, carry):
            def chunk_id(b):
                return wid + NW * (nb * p + b)

            for b in range(nb):
                @pl.when(chunk_id(b) < n_chunks)
                def _(b=b):
                    @pl.when(p > 0)
                    def _():
                        wait_store(b)
                    wait_idx(b)

            for b in range(nb):
                @pl.when(chunk_id(b) < n_chunks)
                def _(b=b):
                    pltpu.async_copy(
                        table_hbm.at[idx_v[b].at[pl.ds(0, ch)]],
                        dst_v[b], sem_g[b])

            for b in range(nb):
                @pl.when(chunk_id(b) < n_chunks)
                def _(b=b):
                    pltpu.make_async_copy(
                        table_hbm.at[idx_v[b].at[pl.ds(0, ch)]],
                        dst_v[b], sem_g[b]).wait()
                    for j in range(1, k):
                        pltpu.async_copy(
                            table_hbm.at[idx_v[b].at[pl.ds(j * ch, ch)]],
                            dst_v[b], sem_g[b], add=True)

            for b in range(nb):
                c = chunk_id(b)

                @pl.when(c < n_chunks)
                def _(b=b, c=c):
                    for j in range(1, k):
                        pltpu.make_async_copy(
                            table_hbm.at[idx_v[b].at[pl.ds(j * ch, ch)]],
                            dst_v[b], sem_g[b]).wait()
                    pltpu.async_copy(dst_v[b], out_hbm.at[pl.ds(c * ch, ch)],
                                     sem_s[b])

                    @pl.when(jnp.logical_and(p + 1 < n_groups,
                                             c + NW * nb < n_chunks))
                    def _():
                        fire_idx(c + NW * nb, b)

            return carry

        lax.fori_loop(0, n_groups, group, 0)

        for b in range(nb):
            @pl.when(wid + NW * b < n_chunks)
            def _(b=b):
                wait_store(b)

    return pl.kernel(
        body,
        out_type=jax.ShapeDtypeStruct((n_chunks * ch, H), jnp.float32),
        mesh=mesh,
        scratch_types=(
            [pltpu.VMEM((k * ch,), jnp.int32) for _ in range(nb)]
            + [pltpu.VMEM((ch, H), jnp.float32) for _ in range(nb)]
            + [pltpu.SemaphoreType.DMA for _ in range(3 * nb)]
        ),
    )


def _chunked_idx(idx, n_chunks, ch=128):
    rows, k = idx.shape
    pad = n_chunks * ch - rows
    if pad:
        idx = jnp.pad(idx, ((0, pad), (0, 0)))
    return idx.reshape(n_chunks, ch, k).transpose(0, 2, 1).reshape(-1)


def kernel(fatoms, fbonds, agraph, bgraph, ascope, W_i, W_h, W_o_w, W_o_b):
    E, Fb = fbonds.shape
    N, Fa = fatoms.shape
    M = ascope.shape[0]
    kb = bgraph.shape[1]
    ka = agraph.shape[1]

    bgraph = bgraph.astype(jnp.int32)
    agraph = agraph.astype(jnp.int32)

    CHA = 64
    ncb = E // CH
    nca = -(-N // CHA)
    idx_b = _chunked_idx(bgraph, ncb, CH)
    idx_a = _chunked_idx(agraph, nca, CHA)

    BM1 = 16000
    binput, message = pl.pallas_call(
        _init_body,
        grid=(E // BM1,),
        in_specs=[pl.BlockSpec((BM1, Fb), lambda i: (i, 0)),
                  pl.BlockSpec((Fb, H), lambda i: (0, 0))],
        out_specs=[pl.BlockSpec((BM1, H), lambda i: (i, 0)),
                   pl.BlockSpec((BM1, H), lambda i: (i, 0))],
        out_shape=[jax.ShapeDtypeStruct((E, H), jnp.bfloat16),
                   jax.ShapeDtypeStruct((E, H), jnp.float32)],
    )(fbonds.astype(jnp.bfloat16), W_i.T.astype(jnp.bfloat16))

    gather_b = _make_gather_sum(kb, ncb, ch=CH, nb=6)
    BM2 = 16000
    update = pl.pallas_call(
        _msg_update_body,
        grid=(E // BM2,),
        in_specs=[pl.BlockSpec((BM2, H), lambda i: (i, 0)),
                  pl.BlockSpec((BM2, H), lambda i: (i, 0)),
                  pl.BlockSpec((H, H), lambda i: (0, 0))],
        out_specs=pl.BlockSpec((BM2, H), lambda i: (i, 0)),
        out_shape=jax.ShapeDtypeStruct((E, H), jnp.float32),
    )
    W_hT = W_h.T
    for _ in range(DEPTH - 1):
        nei = gather_b(message, idx_b)
        message = update(nei, binput, W_hT)

    gather_a = _make_gather_sum(ka, nca, ch=CHA, nb=3)
    a_msg = gather_a(message, idx_a)[:N]

    Wa_T = W_o_w[:, :Fa].T
    Wm_T = W_o_w[:, Fa:].T
    mol_vecs = pl.pallas_call(
        _atom_body,
        grid=(1,),
        in_specs=[pl.BlockSpec((N, Fa), lambda i: (0, 0)),
                  pl.BlockSpec((N, H), lambda i: (0, 0)),
                  pl.BlockSpec((Fa, H), lambda i: (0, 0)),
                  pl.BlockSpec((H, H), lambda i: (0, 0)),
                  pl.BlockSpec((1, H), lambda i: (0, 0))],
        out_specs=pl.BlockSpec((M, H), lambda i: (0, 0)),
        out_shape=jax.ShapeDtypeStruct((M, H), jnp.float32),
    )(fatoms, a_msg, Wa_T, Wm_T, W_o_b.reshape(1, H))
    return mol_vecs
